# baseline (speedup 1.0000x reference)
import functools

import jax
import jax.numpy as jnp
from jax import lax
from jax.experimental import pallas as pl
from jax.experimental.pallas import tpu as pltpu
from jax.experimental.pallas import tpu_sc as plsc

_LANES = 16
_CR = 32


def _rsqrt(x):
    xb = lax.bitcast_convert_type(x, jnp.int32)
    y = lax.bitcast_convert_type(jnp.int32(0x5F3759DF) - (xb >> 1), jnp.float32)
    for _ in range(4):
        y = y * (1.5 - 0.5 * x * y * y)
    return y


def kernel(input_token, token_table, pos_table, ln_gamma, ln_beta):
    B, L = input_token.shape
    V, H = token_table.shape
    nj = H // _LANES

    info = plsc.get_sparse_core_info()
    nw = info.num_cores * info.num_subcores
    lpw = L // nw
    nch = (B * lpw) // _CR
    hpb = lpw // _CR

    mesh = plsc.VectorSubcoreMesh(core_axis_name="c", subcore_axis_name="s")

    @functools.partial(
        pl.kernel,
        out_type=jax.ShapeDtypeStruct((B, L, H), jnp.float32),
        mesh=mesh,
        compiler_params=pltpu.CompilerParams(needs_layout_passes=False),
        scratch_types=[
            pltpu.VMEM((B * lpw,), jnp.int32),
            pltpu.VMEM((lpw, H), jnp.float32),
            pltpu.VMEM((_CR, H), jnp.float32),
            pltpu.VMEM((_CR, H), jnp.float32),
            pltpu.VMEM((_CR, H), jnp.float32),
            pltpu.VMEM((H,), jnp.float32),
            pltpu.VMEM((H,), jnp.float32),
            pltpu.SemaphoreType.DMA,
            pltpu.SemaphoreType.DMA,
            pltpu.SemaphoreType.DMA,
            pltpu.SemaphoreType.DMA,
            pltpu.SemaphoreType.DMA,
            pltpu.SemaphoreType.DMA,
        ],
    )
    def sc_kernel(tok_hbm, table_hbm, pos_hbm, g_hbm, b_hbm, out_hbm,
                  idx_v, pos_v, buf0, buf1, buf2, g_v, bb_v,
                  gsem0, gsem1, gsem2, ssem0, ssem1, ssem2):
        wid = lax.axis_index("s") * info.num_cores + lax.axis_index("c")
        p0 = wid * lpw
        pltpu.sync_copy(pos_hbm.at[pl.ds(p0, lpw)], pos_v)
        pltpu.sync_copy(g_hbm, g_v)
        pltpu.sync_copy(b_hbm, bb_v)
        for b in range(B):
            pltpu.sync_copy(tok_hbm.at[b, pl.ds(p0, lpw)],
                            idx_v.at[pl.ds(b * lpw, lpw)])

        gs = [g_v[pl.ds(j * _LANES, _LANES)] for j in range(nj)]
        bs = [bb_v[pl.ds(j * _LANES, _LANES)] for j in range(nj)]

        bufs = (buf0, buf1, buf2)
        gsems = (gsem0, gsem1, gsem2)
        ssems = (ssem0, ssem1, ssem2)

        def gather(c):
            return pltpu.async_copy(
                table_hbm.at[idx_v.at[pl.ds(c * _CR, _CR)]],
                bufs[c % 3], gsems[c % 3])

        def scatter(c):
            b, h = divmod(c, hpb)
            return pltpu.async_copy(
                bufs[c % 3],
                out_hbm.at[b, pl.ds(p0 + h * _CR, _CR)],
                ssems[c % 3])

        gcps = [None] * nch
        scps = [None] * nch
        gcps[0] = gather(0)
        gcps[1] = gather(1)
        for c in range(nch):
            buf = bufs[c % 3]
            h = c % hpb
            gcps[c].wait()

            def row_body(r, carry, buf=buf, h=h):
                acc = jnp.zeros((_LANES,), jnp.float32)
                acc2 = jnp.zeros((_LANES,), jnp.float32)
                for j in range(nj):
                    sl = pl.ds(j * _LANES, _LANES)
                    v = buf[r, sl] + pos_v[h * _CR + r, sl]
                    buf[r, sl] = v
                    acc = acc + v
                    acc2 = acc2 + v * v
                rcp_h = jnp.float32(1.0 / H)
                mean = jnp.sum(acc) * rcp_h
                var = jnp.sum(acc2) * rcp_h - mean * mean
                s = _rsqrt(var + 1e-5)
                ms = mean * s
                for j in range(nj):
                    sl = pl.ds(j * _LANES, _LANES)
                    y = buf[r, sl] * s - ms
                    buf[r, sl] = y * gs[j] + bs[j]
                return carry

            lax.fori_loop(0, _CR, row_body, 0)
            scps[c] = scatter(c)
            if c + 2 < nch:
                if c > 0:
                    scps[c - 1].wait()
                gcps[c + 2] = gather(c + 2)
        for c in range(max(0, nch - 3), nch):
            scps[c].wait()

    return sc_kernel(input_token, token_table, pos_table, ln_gamma, ln_beta)


# speedup vs baseline: 1.7857x; 1.7857x over previous
import functools

import jax
import jax.numpy as jnp
from jax import lax
from jax.experimental import pallas as pl
from jax.experimental.pallas import tpu as pltpu
from jax.experimental.pallas import tpu_sc as plsc

_CHUNK = 64


def _sc_gather(idx_flat, token_table):
    N = idx_flat.shape[0]
    V, H = token_table.shape

    info = plsc.get_sparse_core_info()
    nw = info.num_cores * info.num_subcores
    rpw = N // nw
    nch = rpw // _CHUNK

    mesh = plsc.VectorSubcoreMesh(core_axis_name="c", subcore_axis_name="s")

    @functools.partial(
        pl.kernel,
        out_type=jax.ShapeDtypeStruct((N, H), jnp.float32),
        mesh=mesh,
        compiler_params=pltpu.CompilerParams(needs_layout_passes=False),
        scratch_types=[
            pltpu.VMEM((rpw,), jnp.int32),
            pltpu.VMEM((_CHUNK, H), jnp.float32),
            pltpu.VMEM((_CHUNK, H), jnp.float32),
            pltpu.SemaphoreType.DMA,
            pltpu.SemaphoreType.DMA,
        ],
    )
    def sc_kernel(idx_hbm, table_hbm, out_hbm, idx_v, buf0, buf1, sem0, sem1):
        wid = lax.axis_index("s") * info.num_cores + lax.axis_index("c")
        base = wid * rpw
        pltpu.sync_copy(idx_hbm.at[pl.ds(base, rpw)], idx_v)
        bufs = (buf0, buf1)
        sems = (sem0, sem1)
        cps = [None] * nch
        cps[0] = pltpu.async_copy(
            table_hbm.at[idx_v.at[pl.ds(0, _CHUNK)]], bufs[0], sems[0])
        for c in range(nch):
            if c + 1 < nch:
                cps[c + 1] = pltpu.async_copy(
                    table_hbm.at[idx_v.at[pl.ds((c + 1) * _CHUNK, _CHUNK)]],
                    bufs[(c + 1) % 2], sems[(c + 1) % 2])
            cps[c].wait()
            pltpu.sync_copy(bufs[c % 2],
                            out_hbm.at[pl.ds(base + c * _CHUNK, _CHUNK)])

    return sc_kernel(idx_flat, token_table)


_R = 256
_NBUF = 4


def _tc_ln(tok_emb, pos_table, ln_gamma, ln_beta):
    N, H = tok_emb.shape
    L = pos_table.shape[0]
    ns = N // _R
    pb = L // _R

    def body(tok_hbm, pos_hbm, g_ref, b_ref, out_hbm, pos_v, inb, outb,
             isem, osem):
        pltpu.sync_copy(pos_hbm, pos_v)
        for i in range(_NBUF):
            pltpu.async_copy(tok_hbm.at[pl.ds(i * _R, _R)], inb.at[i],
                             isem.at[i])

        def step(c, carry):
            k = lax.rem(c, _NBUF)
            pltpu.make_async_copy(tok_hbm.at[pl.ds(c * _R, _R)], inb.at[k],
                                  isem.at[k]).wait()

            @pl.when(c >= _NBUF)
            def _():
                pltpu.make_async_copy(
                    outb.at[k], out_hbm.at[pl.ds((c - _NBUF) * _R, _R)],
                    osem.at[k]).wait()

            p = lax.rem(c, pb)
            x = inb[k] + pos_v[pl.ds(p * _R, _R), :]
            mean = jnp.mean(x, axis=-1, keepdims=True)
            xc = x - mean
            var = jnp.mean(xc * xc, axis=-1, keepdims=True)
            y = xc * lax.rsqrt(var + 1e-5)
            outb[k] = y * g_ref[...] + b_ref[...]
            pltpu.async_copy(outb.at[k], out_hbm.at[pl.ds(c * _R, _R)],
                             osem.at[k])

            @pl.when(c + _NBUF < ns)
            def _():
                pltpu.async_copy(tok_hbm.at[pl.ds((c + _NBUF) * _R, _R)],
                                 inb.at[k], isem.at[k])

            return carry

        lax.fori_loop(0, ns, step, 0)
        for i in range(_NBUF):
            c = ns - _NBUF + i
            k = c % _NBUF
            pltpu.make_async_copy(outb.at[k],
                                  out_hbm.at[pl.ds(c * _R, _R)],
                                  osem.at[k]).wait()

    return pl.pallas_call(
        body,
        in_specs=[
            pl.BlockSpec(memory_space=pltpu.HBM),
            pl.BlockSpec(memory_space=pltpu.HBM),
            pl.BlockSpec(memory_space=pltpu.VMEM),
            pl.BlockSpec(memory_space=pltpu.VMEM),
        ],
        out_specs=pl.BlockSpec(memory_space=pltpu.HBM),
        out_shape=jax.ShapeDtypeStruct((N, H), jnp.float32),
        scratch_shapes=[
            pltpu.VMEM((L, H), jnp.float32),
            pltpu.VMEM((_NBUF, _R, H), jnp.float32),
            pltpu.VMEM((_NBUF, _R, H), jnp.float32),
            pltpu.SemaphoreType.DMA((_NBUF,)),
            pltpu.SemaphoreType.DMA((_NBUF,)),
        ],
    )(tok_emb, pos_table, ln_gamma.reshape(1, H), ln_beta.reshape(1, H))


def kernel(input_token, token_table, pos_table, ln_gamma, ln_beta):
    B, L = input_token.shape
    V, H = token_table.shape

    tok_emb = _sc_gather(input_token.reshape(-1), token_table)
    out = _tc_ln(tok_emb, pos_table, ln_gamma, ln_beta)
    return out.reshape(B, L, H)


# speedup vs baseline: 1.8052x; 1.0109x over previous
import functools

import jax
import jax.numpy as jnp
from jax import lax
from jax.experimental import pallas as pl
from jax.experimental.pallas import tpu as pltpu
from jax.experimental.pallas import tpu_sc as plsc

_CHUNK = 64


def _sc_gather(idx_flat, token_table):
    N = idx_flat.shape[0]
    V, H = token_table.shape

    info = plsc.get_sparse_core_info()
    nw = info.num_cores * info.num_subcores
    rpw = N // nw
    nch = rpw // _CHUNK

    mesh = plsc.VectorSubcoreMesh(core_axis_name="c", subcore_axis_name="s")

    @functools.partial(
        pl.kernel,
        out_type=jax.ShapeDtypeStruct((N, H), jnp.float32),
        mesh=mesh,
        compiler_params=pltpu.CompilerParams(needs_layout_passes=False),
        scratch_types=[
            pltpu.VMEM((rpw,), jnp.int32),
            pltpu.VMEM((_CHUNK, H), jnp.float32),
            pltpu.VMEM((_CHUNK, H), jnp.float32),
            pltpu.SemaphoreType.DMA,
            pltpu.SemaphoreType.DMA,
        ],
    )
    def sc_kernel(idx_hbm, table_hbm, out_hbm, idx_v, buf0, buf1, sem0, sem1):
        wid = lax.axis_index("s") * info.num_cores + lax.axis_index("c")
        base = wid * rpw
        pltpu.sync_copy(idx_hbm.at[pl.ds(base, rpw)], idx_v)
        bufs = (buf0, buf1)
        sems = (sem0, sem1)
        cps = [None] * nch
        cps[0] = pltpu.async_copy(
            table_hbm.at[idx_v.at[pl.ds(0, _CHUNK)]], bufs[0], sems[0])
        for c in range(nch):
            if c + 1 < nch:
                cps[c + 1] = pltpu.async_copy(
                    table_hbm.at[idx_v.at[pl.ds((c + 1) * _CHUNK, _CHUNK)]],
                    bufs[(c + 1) % 2], sems[(c + 1) % 2])
            cps[c].wait()
            pltpu.sync_copy(bufs[c % 2],
                            out_hbm.at[pl.ds(base + c * _CHUNK, _CHUNK)])

    return sc_kernel(idx_flat, token_table)


_R = 256
_NBUF = 4


def _tc_ln(tok_emb, pos_table, ln_gamma, ln_beta):
    N, H = tok_emb.shape
    L = pos_table.shape[0]
    ns = N // _R
    pb = L // _R

    def body(tok_hbm, pos_hbm, g_ref, b_ref, out_hbm, pos_v, inb, outb,
             isem, osem, psem):
        pcp = pltpu.async_copy(pos_hbm, pos_v, psem)
        for i in range(_NBUF):
            pltpu.async_copy(tok_hbm.at[pl.ds(i * _R, _R)], inb.at[i],
                             isem.at[i])
        pcp.wait()

        def step(c, carry):
            k = lax.rem(c, _NBUF)
            pltpu.make_async_copy(tok_hbm.at[pl.ds(c * _R, _R)], inb.at[k],
                                  isem.at[k]).wait()

            @pl.when(c >= _NBUF)
            def _():
                pltpu.make_async_copy(
                    outb.at[k], out_hbm.at[pl.ds((c - _NBUF) * _R, _R)],
                    osem.at[k]).wait()

            p = lax.rem(c, pb)
            x = inb[k] + pos_v[pl.ds(p * _R, _R), :]
            mean = jnp.mean(x, axis=-1, keepdims=True)
            xc = x - mean
            var = jnp.mean(xc * xc, axis=-1, keepdims=True)
            y = xc * lax.rsqrt(var + 1e-5)
            outb[k] = y * g_ref[...] + b_ref[...]
            pltpu.async_copy(outb.at[k], out_hbm.at[pl.ds(c * _R, _R)],
                             osem.at[k])

            @pl.when(c + _NBUF < ns)
            def _():
                pltpu.async_copy(tok_hbm.at[pl.ds((c + _NBUF) * _R, _R)],
                                 inb.at[k], isem.at[k])

            return carry

        lax.fori_loop(0, ns, step, 0)
        for i in range(_NBUF):
            c = ns - _NBUF + i
            k = c % _NBUF
            pltpu.make_async_copy(outb.at[k],
                                  out_hbm.at[pl.ds(c * _R, _R)],
                                  osem.at[k]).wait()

    return pl.pallas_call(
        body,
        in_specs=[
            pl.BlockSpec(memory_space=pltpu.HBM),
            pl.BlockSpec(memory_space=pltpu.HBM),
            pl.BlockSpec(memory_space=pltpu.VMEM),
            pl.BlockSpec(memory_space=pltpu.VMEM),
        ],
        out_specs=pl.BlockSpec(memory_space=pltpu.HBM),
        out_shape=jax.ShapeDtypeStruct((N, H), jnp.float32),
        scratch_shapes=[
            pltpu.VMEM((L, H), jnp.float32),
            pltpu.VMEM((_NBUF, _R, H), jnp.float32),
            pltpu.VMEM((_NBUF, _R, H), jnp.float32),
            pltpu.SemaphoreType.DMA((_NBUF,)),
            pltpu.SemaphoreType.DMA((_NBUF,)),
            pltpu.SemaphoreType.DMA,
        ],
    )(tok_emb, pos_table, ln_gamma.reshape(1, H), ln_beta.reshape(1, H))


def kernel(input_token, token_table, pos_table, ln_gamma, ln_beta):
    B, L = input_token.shape
    V, H = token_table.shape

    tok_emb = _sc_gather(input_token.reshape(-1), token_table)
    out = _tc_ln(tok_emb, pos_table, ln_gamma, ln_beta)
    return out.reshape(B, L, H)


# speedup vs baseline: 1.8494x; 1.0245x over previous
import functools

import jax
import jax.numpy as jnp
from jax import lax
from jax.experimental import pallas as pl
from jax.experimental.pallas import tpu as pltpu
from jax.experimental.pallas import tpu_sc as plsc

_CHUNK = 64


def _sc_gather(idx_flat, token_table):
    N = idx_flat.shape[0]
    V, H = token_table.shape

    info = plsc.get_sparse_core_info()
    nw = info.num_cores * info.num_subcores
    rpw = N // nw
    nch = rpw // _CHUNK

    mesh = plsc.VectorSubcoreMesh(core_axis_name="c", subcore_axis_name="s")

    @functools.partial(
        pl.kernel,
        out_type=jax.ShapeDtypeStruct((N, H), jnp.float32),
        mesh=mesh,
        compiler_params=pltpu.CompilerParams(needs_layout_passes=False),
        scratch_types=[
            pltpu.VMEM((rpw,), jnp.int32),
            pltpu.VMEM((_CHUNK, H), jnp.float32),
            pltpu.VMEM((_CHUNK, H), jnp.float32),
            pltpu.SemaphoreType.DMA,
            pltpu.SemaphoreType.DMA,
        ],
    )
    def sc_kernel(idx_hbm, table_hbm, out_hbm, idx_v, buf0, buf1, sem0, sem1):
        wid = lax.axis_index("s") * info.num_cores + lax.axis_index("c")
        base = wid * rpw
        pltpu.sync_copy(idx_hbm.at[pl.ds(base, rpw)], idx_v)
        bufs = (buf0, buf1)
        sems = (sem0, sem1)
        cps = [None] * nch
        cps[0] = pltpu.async_copy(
            table_hbm.at[idx_v.at[pl.ds(0, _CHUNK)]], bufs[0], sems[0])
        for c in range(nch):
            if c + 1 < nch:
                cps[c + 1] = pltpu.async_copy(
                    table_hbm.at[idx_v.at[pl.ds((c + 1) * _CHUNK, _CHUNK)]],
                    bufs[(c + 1) % 2], sems[(c + 1) % 2])
            cps[c].wait()
            pltpu.sync_copy(bufs[c % 2],
                            out_hbm.at[pl.ds(base + c * _CHUNK, _CHUNK)])

    return sc_kernel(idx_flat, token_table)


_R = 512
_NBUF = 4


def _tc_ln(tok_emb, pos_table, ln_gamma, ln_beta):
    N, H = tok_emb.shape
    L = pos_table.shape[0]
    ns = N // _R
    pb = L // _R

    def body(tok_hbm, pos_hbm, g_ref, b_ref, out_hbm, pos_v, inb, outb,
             isem, osem, psem):
        pcp = pltpu.async_copy(pos_hbm, pos_v, psem)
        for i in range(_NBUF):
            pltpu.async_copy(tok_hbm.at[pl.ds(i * _R, _R)], inb.at[i],
                             isem.at[i])
        pcp.wait()

        def step(c, carry):
            k = lax.rem(c, _NBUF)
            pltpu.make_async_copy(tok_hbm.at[pl.ds(c * _R, _R)], inb.at[k],
                                  isem.at[k]).wait()

            @pl.when(c >= _NBUF)
            def _():
                pltpu.make_async_copy(
                    outb.at[k], out_hbm.at[pl.ds((c - _NBUF) * _R, _R)],
                    osem.at[k]).wait()

            p = lax.rem(c, pb)
            x = inb[k] + pos_v[pl.ds(p * _R, _R), :]
            mean = jnp.mean(x, axis=-1, keepdims=True)
            xc = x - mean
            var = jnp.mean(xc * xc, axis=-1, keepdims=True)
            y = xc * lax.rsqrt(var + 1e-5)
            outb[k] = y * g_ref[...] + b_ref[...]
            pltpu.async_copy(outb.at[k], out_hbm.at[pl.ds(c * _R, _R)],
                             osem.at[k])

            @pl.when(c + _NBUF < ns)
            def _():
                pltpu.async_copy(tok_hbm.at[pl.ds((c + _NBUF) * _R, _R)],
                                 inb.at[k], isem.at[k])

            return carry

        lax.fori_loop(0, ns, step, 0)
        for i in range(_NBUF):
            c = ns - _NBUF + i
            k = c % _NBUF
            pltpu.make_async_copy(outb.at[k],
                                  out_hbm.at[pl.ds(c * _R, _R)],
                                  osem.at[k]).wait()

    return pl.pallas_call(
        body,
        in_specs=[
            pl.BlockSpec(memory_space=pltpu.HBM),
            pl.BlockSpec(memory_space=pltpu.HBM),
            pl.BlockSpec(memory_space=pltpu.VMEM),
            pl.BlockSpec(memory_space=pltpu.VMEM),
        ],
        out_specs=pl.BlockSpec(memory_space=pltpu.HBM),
        out_shape=jax.ShapeDtypeStruct((N, H), jnp.float32),
        scratch_shapes=[
            pltpu.VMEM((L, H), jnp.float32),
            pltpu.VMEM((_NBUF, _R, H), jnp.float32),
            pltpu.VMEM((_NBUF, _R, H), jnp.float32),
            pltpu.SemaphoreType.DMA((_NBUF,)),
            pltpu.SemaphoreType.DMA((_NBUF,)),
            pltpu.SemaphoreType.DMA,
        ],
    )(tok_emb, pos_table, ln_gamma.reshape(1, H), ln_beta.reshape(1, H))


def kernel(input_token, token_table, pos_table, ln_gamma, ln_beta):
    B, L = input_token.shape
    V, H = token_table.shape

    tok_emb = _sc_gather(input_token.reshape(-1), token_table)
    out = _tc_ln(tok_emb, pos_table, ln_gamma, ln_beta)
    return out.reshape(B, L, H)
